# baseline (device time: 37097 ns/iter reference)
import jax
import jax.numpy as jnp
from jax import lax
from jax.experimental import pallas as pl
from jax.experimental.pallas import tpu as pltpu

N_DEV = 16
B, S, C, H = 4, 512, 256, 256
ROWS = B * S
CHUNK = ROWS // N_DEV


def kernel(x, k, Wp):
    def body(x_ref, k_ref, w_ref, out_ref,
             acc_ref, temp_ref, final_ref,
             rs_send, rs_recv, ag_send, ag_recv):
        d = lax.axis_index("i")

        bsem = pltpu.get_barrier_semaphore()
        for r, inc in enumerate((4096, 256, 16, 1)):
            pl.semaphore_signal(
                bsem, inc=inc,
                device_id=(jnp.mod(d + (1 << r), N_DEV),),
                device_id_type=pl.DeviceIdType.MESH,
            )
            pl.semaphore_wait(bsem, inc)

        xv = x_ref[:, :, :]
        kv = k_ref[:, :]
        conv = xv * kv[3:4, :][None, :, :]
        for t in range(3):
            sh = 3 - t
            shifted = jnp.concatenate(
                [jnp.zeros((B, sh, C), jnp.float32), xv[:, : S - sh, :]],
                axis=1,
            )
            conv += shifted * kv[t:t + 1, :][None, :, :]
        a = conv * jax.nn.sigmoid(conv)
        partial = jnp.dot(
            a.reshape(ROWS, C).astype(jnp.bfloat16),
            w_ref[:, :].astype(jnp.bfloat16),
            preferred_element_type=jnp.float32,
        )
        acc_ref[...] = partial.reshape(N_DEV, CHUNK, H).astype(jnp.bfloat16)

        descs = []
        for o in range(1, N_DEV):
            tgt = jnp.mod(d + o, N_DEV)
            r = pltpu.make_async_remote_copy(
                src_ref=acc_ref.at[tgt],
                dst_ref=temp_ref.at[o],
                send_sem=rs_send.at[o],
                recv_sem=rs_recv.at[o],
                device_id=(tgt,),
                device_id_type=pl.DeviceIdType.MESH,
            )
            r.start()
            descs.append(r)
        for r in descs:
            r.wait()

        own = acc_ref[pl.ds(d, 1), :, :].astype(jnp.float32)
        others = jnp.sum(temp_ref[1:, :, :].astype(jnp.float32), axis=0)
        reduced = (own[0] + others).astype(jnp.bfloat16)
        final_ref[pl.ds(d, 1), :, :] = reduced[None, :, :]

        descs2 = []
        for o in range(1, N_DEV):
            tgt = jnp.mod(d + o, N_DEV)
            r = pltpu.make_async_remote_copy(
                src_ref=final_ref.at[d],
                dst_ref=final_ref.at[d],
                send_sem=ag_send.at[o],
                recv_sem=ag_recv.at[o],
                device_id=(tgt,),
                device_id_type=pl.DeviceIdType.MESH,
            )
            r.start()
            descs2.append(r)
        for r in descs2:
            r.wait()

        out_ref[...] = (
            final_ref[:, :, :].astype(jnp.float32).reshape(B, S, H)
        )

    return pl.pallas_call(
        body,
        out_shape=jax.ShapeDtypeStruct((B, S, H), jnp.float32),
        in_specs=[pl.BlockSpec(memory_space=pltpu.VMEM)] * 3,
        out_specs=pl.BlockSpec(memory_space=pltpu.VMEM),
        scratch_shapes=[
            pltpu.VMEM((N_DEV, CHUNK, H), jnp.bfloat16),
            pltpu.VMEM((N_DEV, CHUNK, H), jnp.bfloat16),
            pltpu.VMEM((N_DEV, CHUNK, H), jnp.bfloat16),
            pltpu.SemaphoreType.DMA((N_DEV,)),
            pltpu.SemaphoreType.DMA((N_DEV,)),
            pltpu.SemaphoreType.DMA((N_DEV,)),
            pltpu.SemaphoreType.DMA((N_DEV,)),
        ],
        compiler_params=pltpu.CompilerParams(collective_id=0),
    )(x, k, Wp)


# device time: 32830 ns/iter; 1.1300x vs baseline; 1.1300x over previous
import jax
import jax.numpy as jnp
from jax import lax
from jax.experimental import pallas as pl
from jax.experimental.pallas import tpu as pltpu

N_DEV = 16
B, S, C, H = 4, 512, 256, 256
ROWS = B * S
CHUNK = ROWS // N_DEV


def kernel(x, k, Wp):
    def body(x_ref, k_ref, w_ref, out_ref,
             acc_ref, temp_ref, final_ref,
             rs_send, rs_recv, ag_send, ag_recv):
        d = lax.axis_index("i")

        bsem = pltpu.get_barrier_semaphore()
        for o in range(1, N_DEV):
            pl.semaphore_signal(
                bsem, inc=1,
                device_id=(jnp.mod(d + o, N_DEV),),
                device_id_type=pl.DeviceIdType.MESH,
            )

        xv = x_ref[:, :, :]
        kv = k_ref[:, :]
        conv = xv * kv[3:4, :][None, :, :]
        for t in range(3):
            sh = 3 - t
            shifted = jnp.concatenate(
                [jnp.zeros((B, sh, C), jnp.float32), xv[:, : S - sh, :]],
                axis=1,
            )
            conv += shifted * kv[t:t + 1, :][None, :, :]
        a = conv * jax.nn.sigmoid(conv)
        partial = jnp.dot(
            a.reshape(ROWS, C).astype(jnp.bfloat16),
            w_ref[:, :].astype(jnp.bfloat16),
            preferred_element_type=jnp.float32,
        )
        acc_ref[...] = partial.reshape(N_DEV, CHUNK, H).astype(jnp.bfloat16)

        pl.semaphore_wait(bsem, N_DEV - 1)

        descs = []
        for o in range(1, N_DEV):
            tgt = jnp.mod(d + o, N_DEV)
            r = pltpu.make_async_remote_copy(
                src_ref=acc_ref.at[tgt],
                dst_ref=temp_ref.at[o],
                send_sem=rs_send.at[o],
                recv_sem=rs_recv.at[o],
                device_id=(tgt,),
                device_id_type=pl.DeviceIdType.MESH,
            )
            r.start()
            descs.append(r)
        for r in descs:
            r.wait()

        own = acc_ref[pl.ds(d, 1), :, :].astype(jnp.float32)
        others = jnp.sum(temp_ref[1:, :, :].astype(jnp.float32), axis=0)
        reduced = (own[0] + others).astype(jnp.bfloat16)
        final_ref[pl.ds(d, 1), :, :] = reduced[None, :, :]

        descs2 = []
        for o in range(1, N_DEV):
            tgt = jnp.mod(d + o, N_DEV)
            r = pltpu.make_async_remote_copy(
                src_ref=final_ref.at[d],
                dst_ref=final_ref.at[d],
                send_sem=ag_send.at[o],
                recv_sem=ag_recv.at[o],
                device_id=(tgt,),
                device_id_type=pl.DeviceIdType.MESH,
            )
            r.start()
            descs2.append(r)
        for r in descs2:
            r.wait()

        out_ref[...] = (
            final_ref[:, :, :].astype(jnp.float32).reshape(B, S, H)
        )

    return pl.pallas_call(
        body,
        out_shape=jax.ShapeDtypeStruct((B, S, H), jnp.float32),
        in_specs=[pl.BlockSpec(memory_space=pltpu.VMEM)] * 3,
        out_specs=pl.BlockSpec(memory_space=pltpu.VMEM),
        scratch_shapes=[
            pltpu.VMEM((N_DEV, CHUNK, H), jnp.bfloat16),
            pltpu.VMEM((N_DEV, CHUNK, H), jnp.bfloat16),
            pltpu.VMEM((N_DEV, CHUNK, H), jnp.bfloat16),
            pltpu.SemaphoreType.DMA((N_DEV,)),
            pltpu.SemaphoreType.DMA((N_DEV,)),
            pltpu.SemaphoreType.DMA((N_DEV,)),
            pltpu.SemaphoreType.DMA((N_DEV,)),
        ],
        compiler_params=pltpu.CompilerParams(collective_id=0),
    )(x, k, Wp)
